# baseline (speedup 1.0000x reference)
import functools

import numpy as np
import jax
import jax.numpy as jnp
from jax import lax
from jax.experimental import pallas as pl
from jax.experimental.pallas import tpu as pltpu
from jax.experimental.pallas import tpu_sc as plsc

_B = 4
_N = 16384
_C = 6
_G = 512
_K = 32
_BIG = np.float32(1e10)
_HUGE = np.float32(1e30)
_IMAX = np.int32(2**31 - 1)
_NC = 4096
_NH = _NC // 2
_NCHUNKS = _N // _NC
_NCAND = _NCHUNKS * _K


def _fps_body(len_ref, x_ref, y_ref, z_ref, cen_ref):
    X = x_ref[...]
    Y = y_ref[...]
    Z = z_ref[...]
    rows = lax.broadcasted_iota(jnp.int32, (_B, 128, 128), 1)
    cols = lax.broadcasted_iota(jnp.int32, (_B, 128, 128), 2)
    lin = rows * 128 + cols
    bidx = lax.broadcasted_iota(jnp.int32, (_B, 1, 1), 0)
    Lv = jnp.zeros((_B, 1, 1), jnp.int32)
    for bb in range(_B):
        Lv = jnp.where(bidx == bb, len_ref[bb], Lv)
    valid = lin < Lv

    cx0 = X[:, 0:1, 0:1]
    cy0 = Y[:, 0:1, 0:1]
    cz0 = Z[:, 0:1, 0:1]
    cen_ref[:, 0:1, :] = jnp.concatenate([cx0, cy0, cz0], axis=2)
    dmin0 = jnp.where(valid, _BIG, -_BIG)

    def fps_step(i, st):
        dmin, cx, cy, cz = st
        d = (X - cx) ** 2 + (Y - cy) ** 2 + (Z - cz) ** 2
        d = jnp.where(valid, d, -_BIG)
        dmin = jnp.minimum(dmin, d)
        m = jnp.max(dmin, axis=(1, 2), keepdims=True)
        sel = jnp.min(jnp.where(dmin == m, lin, _IMAX), axis=(1, 2),
                      keepdims=True)
        onehot = lin == sel
        zf = np.float32(0.0)
        ncx = jnp.sum(jnp.where(onehot, X, zf), axis=(1, 2), keepdims=True)
        ncy = jnp.sum(jnp.where(onehot, Y, zf), axis=(1, 2), keepdims=True)
        ncz = jnp.sum(jnp.where(onehot, Z, zf), axis=(1, 2), keepdims=True)
        cen_ref[:, pl.ds(i, 1), :] = jnp.concatenate([ncx, ncy, ncz], axis=2)
        return (dmin, ncx, ncy, ncz)

    lax.fori_loop(1, _G, fps_step, (dmin0, cx0, cy0, cz0), unroll=False)


def _knn_body(len_ref, xt_ref, cen_ref, idx_ref, m_ref, i_ref, ld_ref,
              cd_ref, ci_ref):
    b = pl.program_id(0)
    L = len_ref[b]
    ccx = cen_ref[0, :, 0:1]
    ccy = cen_ref[0, :, 1:2]
    ccz = cen_ref[0, :, 2:3]
    gbase = b * _N
    slot_iota = lax.broadcasted_iota(jnp.int32, (1, _NCAND), 1)

    def chunk_body(k, _):
        xa = xt_ref[0, 0:1, pl.ds(k * _NC, _NH)]
        ya = xt_ref[0, 1:2, pl.ds(k * _NC, _NH)]
        za = xt_ref[0, 2:3, pl.ds(k * _NC, _NH)]
        xb = xt_ref[0, 0:1, pl.ds(k * _NC + _NH, _NH)]
        yb = xt_ref[0, 1:2, pl.ds(k * _NC + _NH, _NH)]
        zb = xt_ref[0, 2:3, pl.ds(k * _NC + _NH, _NH)]
        lina = gbase + k * _NC + lax.broadcasted_iota(jnp.int32, (1, _NH), 1)
        linb = lina + _NH
        vlim = gbase + L
        Da = (ccx - xa) ** 2 + (ccy - ya) ** 2 + (ccz - za) ** 2
        Da = jnp.where(lina < vlim, Da, _BIG)
        Db = (ccx - xb) ** 2 + (ccy - yb) ** 2 + (ccz - zb) ** 2
        Db = jnp.where(linb < vlim, Db, _BIG)
        ltb = Db < Da
        m_ref[...] = jnp.where(ltb, Db, Da)
        i_ref[...] = jnp.where(ltb, linb, lina)
        ld_ref[...] = jnp.where(ltb, Da, Db)
        linsum = lina + linb

        def ext(t, _):
            Mv = m_ref[...]
            Iv = i_ref[...]
            m = jnp.min(Mv, axis=1, keepdims=True)
            eq = Mv == m
            sel = jnp.min(jnp.where(eq, Iv, _IMAX), axis=1, keepdims=True)
            msk = Iv == sel
            m_ref[...] = jnp.where(msk, ld_ref[...], Mv)
            i_ref[...] = jnp.where(msk, linsum - Iv, Iv)
            ld_ref[...] = jnp.where(msk, _HUGE, ld_ref[...])
            oh = slot_iota == (k * _K + t)
            cd_ref[...] = jnp.where(oh, m, cd_ref[...])
            ci_ref[...] = jnp.where(oh, sel, ci_ref[...])
            return 0

        lax.fori_loop(0, _K, ext, 0, unroll=False)
        return 0

    lax.fori_loop(0, _NCHUNKS, chunk_body, 0, unroll=False)

    idx_ref[0] = jnp.zeros((_G, _K), jnp.int32)
    koh = lax.broadcasted_iota(jnp.int32, (1, _K), 1)

    def mext(t, _):
        cd = cd_ref[...]
        ci = ci_ref[...]
        m = jnp.min(cd, axis=1, keepdims=True)
        eq = cd == m
        sel = jnp.min(jnp.where(eq, ci, _IMAX), axis=1, keepdims=True)
        cd_ref[...] = jnp.where(ci == sel, _HUGE, cd)
        idx_ref[0] = jnp.where(koh == t, sel, idx_ref[0])
        return 0

    lax.fori_loop(0, _K, mext, 0, unroll=False)


def _fps_knn(points, lengths):
    xyz = points[:, :, :3]
    Xp = xyz[:, :, 0].reshape(_B, 128, 128)
    Yp = xyz[:, :, 1].reshape(_B, 128, 128)
    Zp = xyz[:, :, 2].reshape(_B, 128, 128)
    xt = jnp.transpose(xyz, (0, 2, 1))
    lengths32 = lengths.astype(jnp.int32)

    cen = pl.pallas_call(
        _fps_body,
        in_specs=[
            pl.BlockSpec(memory_space=pltpu.SMEM),
            pl.BlockSpec((_B, 128, 128), lambda: (0, 0, 0)),
            pl.BlockSpec((_B, 128, 128), lambda: (0, 0, 0)),
            pl.BlockSpec((_B, 128, 128), lambda: (0, 0, 0)),
        ],
        out_specs=pl.BlockSpec((_B, _G, 3), lambda: (0, 0, 0)),
        out_shape=jax.ShapeDtypeStruct((_B, _G, 3), jnp.float32),
    )(lengths32, Xp, Yp, Zp)

    idx = pl.pallas_call(
        _knn_body,
        grid=(_B,),
        in_specs=[
            pl.BlockSpec(memory_space=pltpu.SMEM),
            pl.BlockSpec((1, 3, _N), lambda b: (b, 0, 0)),
            pl.BlockSpec((1, _G, 3), lambda b: (b, 0, 0)),
        ],
        out_specs=pl.BlockSpec((1, _G, _K), lambda b: (b, 0, 0)),
        out_shape=jax.ShapeDtypeStruct((_B, _G, _K), jnp.int32),
        scratch_shapes=[
            pltpu.VMEM((_G, _NH), jnp.float32),
            pltpu.VMEM((_G, _NH), jnp.int32),
            pltpu.VMEM((_G, _NH), jnp.float32),
            pltpu.VMEM((_G, _NCAND), jnp.float32),
            pltpu.VMEM((_G, _NCAND), jnp.int32),
        ],
    )(lengths32, xt, cen)
    return cen, idx


_NW = 32
_RPW = (_B * _G * _K) // _NW
_FPW = _RPW * _C
_GPW = _RPW // _K


def _gather_body(pts_hbm, idx_hbm, cen_hbm, out_hbm, tab_v, idx_v, cen_v, out_v):
    c = lax.axis_index("c")
    s = lax.axis_index("s")
    wid = s * 2 + c
    b = wid // (_NW // _B)

    pltpu.sync_copy(pts_hbm.at[pl.ds(b * _N * _C, _N * _C)], tab_v)
    pltpu.sync_copy(idx_hbm.at[pl.ds(wid * _RPW, _RPW)], idx_v)
    pltpu.sync_copy(cen_hbm.at[pl.ds(wid * _GPW * 3, _GPW * 3)], cen_v)

    tbase = b * (_N * _C)

    def sub_body(v, _):
        lane = lax.iota(jnp.int32, 16)
        pos = v * 16 + lane
        r = (pos * 43691) >> 18
        ch = pos - r * _C
        isxyz = -((ch - 3) >> 31)
        maskf = isxyz.astype(jnp.float32)
        chmin = ch * isxyz + 2 * (1 - isxyz)
        cidx = (r >> 5) * 3 + chmin
        gidx = plsc.load_gather(idx_v, [r])
        val = plsc.load_gather(tab_v, [gidx * _C + ch - tbase])
        sub = plsc.load_gather(cen_v, [cidx])
        out_v[pl.ds(v * 16, 16)] = val - sub * maskf
        return 0

    lax.fori_loop(0, _FPW // 16, sub_body, 0, unroll=False)
    pltpu.sync_copy(out_v, out_hbm.at[pl.ds(wid * _FPW, _FPW)])


def _gather_groups(points, idx, cen):
    ptsf = points.reshape(_B * _N * _C)
    idxf = idx.reshape(_B * _G * _K)
    cenf = cen.reshape(_B * _G * 3)

    mesh = plsc.VectorSubcoreMesh(core_axis_name="c", subcore_axis_name="s")
    k = functools.partial(
        pl.kernel,
        mesh=mesh,
        compiler_params=pltpu.CompilerParams(needs_layout_passes=False),
        out_type=jax.ShapeDtypeStruct((_B * _G * _K * _C,), jnp.float32),
        scratch_types=[
            pltpu.VMEM((_N * _C,), jnp.float32),
            pltpu.VMEM((_RPW,), jnp.int32),
            pltpu.VMEM((_GPW * 3,), jnp.float32),
            pltpu.VMEM((_FPW,), jnp.float32),
        ],
    )(_gather_body)
    out = k(ptsf, idxf, cenf)
    return out.reshape(_B, _G, _K, _C)


def kernel(points, lengths):
    cen, idx = _fps_knn(points.astype(jnp.float32), lengths)
    groups = _gather_groups(points, idx, cen)
    return groups, cen


# speedup vs baseline: 1.0361x; 1.0361x over previous
import functools

import numpy as np
import jax
import jax.numpy as jnp
from jax import lax
from jax.experimental import pallas as pl
from jax.experimental.pallas import tpu as pltpu
from jax.experimental.pallas import tpu_sc as plsc

_B = 4
_N = 16384
_C = 6
_G = 512
_K = 32
_BIG = np.float32(1e10)
_HUGE = np.float32(1e30)
_IMAX = np.int32(2**31 - 1)
_NC = 4096
_NH = _NC // 2
_NCHUNKS = _N // _NC
_NCAND = _NCHUNKS * _K


def _fps_body(len_ref, x_ref, y_ref, z_ref, cen_ref):
    X = x_ref[...]
    Y = y_ref[...]
    Z = z_ref[...]
    rows = lax.broadcasted_iota(jnp.int32, (_B, 128, 128), 1)
    cols = lax.broadcasted_iota(jnp.int32, (_B, 128, 128), 2)
    lin = rows * 128 + cols
    bidx = lax.broadcasted_iota(jnp.int32, (_B, 1, 1), 0)
    Lv = jnp.zeros((_B, 1, 1), jnp.int32)
    for bb in range(_B):
        Lv = jnp.where(bidx == bb, len_ref[bb], Lv)
    valid = lin < Lv

    cx0 = X[:, 0:1, 0:1]
    cy0 = Y[:, 0:1, 0:1]
    cz0 = Z[:, 0:1, 0:1]
    cen_ref[:, 0:1, :] = jnp.concatenate([cx0, cy0, cz0], axis=2)
    dmin0 = jnp.where(valid, _BIG, -_BIG)

    def fps_step(i, st):
        dmin, cx, cy, cz = st
        d = (X - cx) ** 2 + (Y - cy) ** 2 + (Z - cz) ** 2
        d = jnp.where(valid, d, -_BIG)
        dmin = jnp.minimum(dmin, d)
        m = jnp.max(dmin, axis=(1, 2), keepdims=True)
        sel = jnp.min(jnp.where(dmin == m, lin, _IMAX), axis=(1, 2),
                      keepdims=True)
        onehot = lin == sel
        zf = np.float32(0.0)
        ncx = jnp.sum(jnp.where(onehot, X, zf), axis=(1, 2), keepdims=True)
        ncy = jnp.sum(jnp.where(onehot, Y, zf), axis=(1, 2), keepdims=True)
        ncz = jnp.sum(jnp.where(onehot, Z, zf), axis=(1, 2), keepdims=True)
        cen_ref[:, pl.ds(i, 1), :] = jnp.concatenate([ncx, ncy, ncz], axis=2)
        return (dmin, ncx, ncy, ncz)

    lax.fori_loop(1, _G, fps_step, (dmin0, cx0, cy0, cz0), unroll=False)


def _knn_body(len_ref, xt_ref, cen_ref, idx_ref, m_ref, i_ref, ld_ref, li_ref,
              cd_ref, ci_ref):
    b = pl.program_id(0)
    L = len_ref[b]
    ccx = cen_ref[0, :, 0:1]
    ccy = cen_ref[0, :, 1:2]
    ccz = cen_ref[0, :, 2:3]
    gbase = b * _N
    slot_iota = lax.broadcasted_iota(jnp.int32, (1, _NCAND), 1)

    def chunk_body(k, _):
        xa = xt_ref[0, 0:1, pl.ds(k * _NC, _NH)]
        ya = xt_ref[0, 1:2, pl.ds(k * _NC, _NH)]
        za = xt_ref[0, 2:3, pl.ds(k * _NC, _NH)]
        xb = xt_ref[0, 0:1, pl.ds(k * _NC + _NH, _NH)]
        yb = xt_ref[0, 1:2, pl.ds(k * _NC + _NH, _NH)]
        zb = xt_ref[0, 2:3, pl.ds(k * _NC + _NH, _NH)]
        lina = gbase + k * _NC + lax.broadcasted_iota(jnp.int32, (1, _NH), 1)
        linb = lina + _NH
        vlim = gbase + L
        Da = (ccx - xa) ** 2 + (ccy - ya) ** 2 + (ccz - za) ** 2
        Da = jnp.where(lina < vlim, Da, _BIG)
        Db = (ccx - xb) ** 2 + (ccy - yb) ** 2 + (ccz - zb) ** 2
        Db = jnp.where(linb < vlim, Db, _BIG)
        ltb = Db < Da
        m_ref[...] = jnp.where(ltb, Db, Da)
        i_ref[...] = jnp.where(ltb, linb, lina)
        ld_ref[...] = jnp.where(ltb, Da, Db)
        li_ref[...] = jnp.where(ltb, lina, linb)

        def ext(t, _):
            Mv = m_ref[...]
            Iv = i_ref[...]
            m = jnp.min(Mv, axis=1, keepdims=True)
            eq = Mv == m
            sel = jnp.min(jnp.where(eq, Iv, _IMAX), axis=1, keepdims=True)
            msk = Iv == sel
            m_ref[...] = jnp.where(msk, ld_ref[...], Mv)
            i_ref[...] = jnp.where(msk, li_ref[...], Iv)
            ld_ref[...] = jnp.where(msk, _HUGE, ld_ref[...])
            oh = slot_iota == (k * _K + t)
            cd_ref[...] = jnp.where(oh, m, cd_ref[...])
            ci_ref[...] = jnp.where(oh, sel, ci_ref[...])
            return 0

        lax.fori_loop(0, _K, ext, 0, unroll=False)
        return 0

    lax.fori_loop(0, _NCHUNKS, chunk_body, 0, unroll=False)

    idx_ref[0] = jnp.zeros((_G, _K), jnp.int32)
    koh = lax.broadcasted_iota(jnp.int32, (1, _K), 1)

    def mext(t, _):
        cd = cd_ref[...]
        ci = ci_ref[...]
        m = jnp.min(cd, axis=1, keepdims=True)
        eq = cd == m
        sel = jnp.min(jnp.where(eq, ci, _IMAX), axis=1, keepdims=True)
        cd_ref[...] = jnp.where(ci == sel, _HUGE, cd)
        idx_ref[0] = jnp.where(koh == t, sel, idx_ref[0])
        return 0

    lax.fori_loop(0, _K, mext, 0, unroll=False)


def _fps_knn(points, lengths):
    xyz = points[:, :, :3]
    Xp = xyz[:, :, 0].reshape(_B, 128, 128)
    Yp = xyz[:, :, 1].reshape(_B, 128, 128)
    Zp = xyz[:, :, 2].reshape(_B, 128, 128)
    xt = jnp.transpose(xyz, (0, 2, 1))
    lengths32 = lengths.astype(jnp.int32)

    cen = pl.pallas_call(
        _fps_body,
        in_specs=[
            pl.BlockSpec(memory_space=pltpu.SMEM),
            pl.BlockSpec((_B, 128, 128), lambda: (0, 0, 0)),
            pl.BlockSpec((_B, 128, 128), lambda: (0, 0, 0)),
            pl.BlockSpec((_B, 128, 128), lambda: (0, 0, 0)),
        ],
        out_specs=pl.BlockSpec((_B, _G, 3), lambda: (0, 0, 0)),
        out_shape=jax.ShapeDtypeStruct((_B, _G, 3), jnp.float32),
    )(lengths32, Xp, Yp, Zp)

    idx = pl.pallas_call(
        _knn_body,
        grid=(_B,),
        in_specs=[
            pl.BlockSpec(memory_space=pltpu.SMEM),
            pl.BlockSpec((1, 3, _N), lambda b: (b, 0, 0)),
            pl.BlockSpec((1, _G, 3), lambda b: (b, 0, 0)),
        ],
        out_specs=pl.BlockSpec((1, _G, _K), lambda b: (b, 0, 0)),
        out_shape=jax.ShapeDtypeStruct((_B, _G, _K), jnp.int32),
        scratch_shapes=[
            pltpu.VMEM((_G, _NH), jnp.float32),
            pltpu.VMEM((_G, _NH), jnp.int32),
            pltpu.VMEM((_G, _NH), jnp.float32),
            pltpu.VMEM((_G, _NH), jnp.int32),
            pltpu.VMEM((_G, _NCAND), jnp.float32),
            pltpu.VMEM((_G, _NCAND), jnp.int32),
        ],
    )(lengths32, xt, cen)
    return cen, idx


_NW = 32
_RPW = (_B * _G * _K) // _NW
_FPW = _RPW * _C
_GPW = _RPW // _K


def _gather_body(pts_hbm, idx_hbm, cen_hbm, out_hbm, tab_v, idx_v, cen_v, out_v):
    c = lax.axis_index("c")
    s = lax.axis_index("s")
    wid = s * 2 + c
    b = wid // (_NW // _B)

    pltpu.sync_copy(pts_hbm.at[pl.ds(b * _N * _C, _N * _C)], tab_v)
    pltpu.sync_copy(idx_hbm.at[pl.ds(wid * _RPW, _RPW)], idx_v)
    pltpu.sync_copy(cen_hbm.at[pl.ds(wid * _GPW * 3, _GPW * 3)], cen_v)

    tbase = b * (_N * _C)

    def sub_body(v, _):
        lane = lax.iota(jnp.int32, 16)
        pos = v * 16 + lane
        r = (pos * 43691) >> 18
        ch = pos - r * _C
        isxyz = -((ch - 3) >> 31)
        maskf = isxyz.astype(jnp.float32)
        chmin = ch * isxyz + 2 * (1 - isxyz)
        cidx = (r >> 5) * 3 + chmin
        gidx = plsc.load_gather(idx_v, [r])
        val = plsc.load_gather(tab_v, [gidx * _C + ch - tbase])
        sub = plsc.load_gather(cen_v, [cidx])
        out_v[pl.ds(v * 16, 16)] = val - sub * maskf
        return 0

    lax.fori_loop(0, _FPW // 16, sub_body, 0, unroll=False)
    pltpu.sync_copy(out_v, out_hbm.at[pl.ds(wid * _FPW, _FPW)])


def _gather_groups(points, idx, cen):
    ptsf = points.reshape(_B * _N * _C)
    idxf = idx.reshape(_B * _G * _K)
    cenf = cen.reshape(_B * _G * 3)

    mesh = plsc.VectorSubcoreMesh(core_axis_name="c", subcore_axis_name="s")
    k = functools.partial(
        pl.kernel,
        mesh=mesh,
        compiler_params=pltpu.CompilerParams(needs_layout_passes=False),
        out_type=jax.ShapeDtypeStruct((_B * _G * _K * _C,), jnp.float32),
        scratch_types=[
            pltpu.VMEM((_N * _C,), jnp.float32),
            pltpu.VMEM((_RPW,), jnp.int32),
            pltpu.VMEM((_GPW * 3,), jnp.float32),
            pltpu.VMEM((_FPW,), jnp.float32),
        ],
    )(_gather_body)
    out = k(ptsf, idxf, cenf)
    return out.reshape(_B, _G, _K, _C)


def kernel(points, lengths):
    cen, idx = _fps_knn(points.astype(jnp.float32), lengths)
    groups = _gather_groups(points, idx, cen)
    return groups, cen
